# baseline (speedup 1.0000x reference)
import functools

import jax
import jax.numpy as jnp
from jax import lax
from jax.experimental import pallas as pl
from jax.experimental.pallas import tpu as pltpu
from jax.experimental.pallas import tpu_sc as plsc

B = 1024
NUSERS = 16384
NNEWS = 100001
DIM = 64
TITLE = 768
NHOP = 2
NMEM = 32
KGE_W = 0.01
L2_W = 1e-07

NC = 2
NS = 16
NW = NC * NS
SEG = 4
SB = B // SEG
UPT = SB // NW
NCAND = 5
CPT = NCAND * UPT
RPT = UPT * NMEM
CCH = CPT // NCAND

_ROLES = (
    ('ent', (1 * 3 + 0) * NMEM),
    ('ent', (2 * 3 + 0) * NMEM),
    ('rel', (0 * 3 + 1) * NMEM),
    ('rel', (1 * 3 + 1) * NMEM),
    ('rel', (2 * 3 + 1) * NMEM),
    ('ent', (0 * 3 + 2) * NMEM),
    ('ent', (1 * 3 + 2) * NMEM),
    ('ent', (2 * 3 + 2) * NMEM),
)

BB = 64
GRID = SB // BB
_GU = BB // UPT

ECH = 128
NECH = RPT // ECH
NCH = 64
NNCH = RPT // NCH


def _k1_body(uidx_hbm, ripple_hbm, ent_hbm, rel_hbm,
             out_small, out_nidx,
             uidx_v, rs_v, ridx_v, nidx_v, ebuf, sem_g, sem_w):
    wid = lax.axis_index("s") * NC + lax.axis_index("c")
    ubase = pl.multiple_of(wid * UPT, 8)

    pltpu.sync_copy(uidx_hbm.at[pl.ds(ubase, UPT)], uidx_v)
    pltpu.async_copy(ripple_hbm.at[uidx_v], rs_v, sem_g).wait()

    iota = lax.iota(jnp.int32, 16)
    def build(u, carry):
        uvec = jnp.full((16,), u, jnp.int32)
        for c in range(2):
            nidx_v[pl.ds(c * (RPT // 2) + u * 16, 16)] = \
                plsc.load_gather(rs_v, [uvec, 2 * iota + c])
            for p, (_, off) in enumerate(_ROLES):
                ridx_v[p, pl.ds(u * NMEM + c * 16, 16)] = \
                    rs_v[u, pl.ds(off + c * 16, 16)]
        return carry

    lax.fori_loop(0, UPT, build, 0)

    gu = _GU
    lob = pl.multiple_of(
        (wid // gu) * (BB * NMEM) + (wid % gu) * (UPT * NMEM // 2), 8)
    pltpu.sync_copy(nidx_v.at[pl.ds(0, RPT // 2)],
                    out_nidx.at[pl.ds(lob, RPT // 2)])
    pltpu.sync_copy(nidx_v.at[pl.ds(RPT // 2, RPT // 2)],
                    out_nidx.at[pl.ds(lob + BB * NMEM // 2, RPT // 2)])

    seq = [(p, ch) for p in range(8) for ch in range(NECH)]
    gd = {}
    wd = {}
    for j, (p, ch) in enumerate(seq):
        b = j % 2
        if j >= 2:
            wd[j - 2].wait()
        src = ent_hbm if _ROLES[p][0] == 'ent' else rel_hbm
        gd[j] = pltpu.async_copy(
            src.at[ridx_v.at[p, pl.ds(ch * ECH, ECH)]], ebuf.at[b], sem_g)
        if j >= 1:
            pj, pch = seq[j - 1]
            gd[j - 1].wait()
            wd[j - 1] = pltpu.async_copy(
                ebuf.at[(j - 1) % 2],
                out_small.at[pj, pl.ds(wid * RPT + pch * ECH, ECH)], sem_w)
    j = len(seq) - 1
    gd[j].wait()
    wd[j] = pltpu.async_copy(
        ebuf.at[j % 2],
        out_small.at[seq[j][0], pl.ds(wid * RPT + seq[j][1] * ECH, ECH)],
        sem_w)
    wd[j - 1].wait()
    wd[j].wait()


def _k2_body(nidx_hbm, cand_hbm, news_hbm, out_news, out_cand,
             nidx_v, cidx_v, nbuf, sem_g, sem_w):
    wid = lax.axis_index("s") * NC + lax.axis_index("c")
    nbase = pl.multiple_of(wid * RPT, 8)
    cbase = pl.multiple_of(wid * CPT, 8)

    pltpu.sync_copy(nidx_hbm.at[pl.ds(nbase, RPT)], nidx_v)
    pltpu.sync_copy(cand_hbm.at[pl.ds(cbase, CPT)], cidx_v)

    seq = [('n', ch) for ch in range(NNCH)] + [('c', ch) for ch in range(NCAND)]

    def fire(j):
        kind, ch = seq[j]
        b = j % 2
        if kind == 'n':
            return pltpu.async_copy(
                news_hbm.at[nidx_v.at[pl.ds(ch * NCH, NCH)]],
                nbuf.at[b], sem_g)
        return pltpu.async_copy(
            news_hbm.at[cidx_v.at[pl.ds(ch * CCH, CCH)]],
            nbuf.at[b, pl.ds(0, CCH)], sem_g)

    def drain(j):
        kind, ch = seq[j]
        b = j % 2
        if kind == 'n':
            return pltpu.async_copy(
                nbuf.at[b], out_news.at[pl.ds(nbase + ch * NCH, NCH)], sem_w)
        return pltpu.async_copy(
            nbuf.at[b, pl.ds(0, CCH)],
            out_cand.at[pl.ds(cbase + ch * CCH, CCH)], sem_w)

    gd = {}
    wd = {}
    for j in range(len(seq)):
        if j >= 2:
            wd[j - 2].wait()
        gd[j] = fire(j)
        if j >= 1:
            gd[j - 1].wait()
            wd[j - 1] = drain(j - 1)
    j = len(seq) - 1
    gd[j].wait()
    wd[j] = drain(j)
    wd[j - 1].wait()
    wd[j].wait()


@functools.cache
def _get_k1():
    return functools.partial(
        pl.kernel,
        out_type=(
            jax.ShapeDtypeStruct((8, SB * NMEM, DIM), jnp.float32),
            jax.ShapeDtypeStruct((SB * NMEM,), jnp.int32),
        ),
        mesh=plsc.VectorSubcoreMesh(core_axis_name="c", subcore_axis_name="s",
                                    num_cores=NC, num_subcores=NS),
        scratch_types=[
            pltpu.VMEM((UPT,), jnp.int32),
            pltpu.VMEM((UPT, 9 * NMEM), jnp.int32),
            pltpu.VMEM((8, RPT), jnp.int32),
            pltpu.VMEM((RPT,), jnp.int32),
            pltpu.VMEM((2, ECH, DIM), jnp.float32),
            pltpu.SemaphoreType.DMA,
            pltpu.SemaphoreType.DMA,
        ],
        compiler_params=pltpu.CompilerParams(use_tc_tiling_on_sc=False,
                                             needs_layout_passes=False),
    )(_k1_body)


@functools.cache
def _get_k2():
    return functools.partial(
        pl.kernel,
        out_type=(
            jax.ShapeDtypeStruct((SB * NMEM, TITLE), jnp.float32),
            jax.ShapeDtypeStruct((SB * NCAND, TITLE), jnp.float32),
        ),
        mesh=plsc.VectorSubcoreMesh(core_axis_name="c", subcore_axis_name="s",
                                    num_cores=NC, num_subcores=NS),
        scratch_types=[
            pltpu.VMEM((RPT,), jnp.int32),
            pltpu.VMEM((CPT,), jnp.int32),
            pltpu.VMEM((2, NCH, TITLE), jnp.float32),
            pltpu.SemaphoreType.DMA,
            pltpu.SemaphoreType.DMA,
        ],
        compiler_params=pltpu.CompilerParams(use_tc_tiling_on_sc=True),
    )(_k2_body)


NF = NMEM // 2


def _tc_body(newsA_ref, newsB_ref, cand_ref, smallf_ref, labels_ref, wn_ref,
             bn_ref, wt_ref, scores_ref, acc_ref):
    i = pl.program_id(0)
    wn = wn_ref[...]
    bn = bn_ref[...]
    wt = wt_ref[...]

    ha = jnp.tanh(
        jnp.dot(newsA_ref[...], wn, preferred_element_type=jnp.float32) + bn)
    hb = jnp.tanh(
        jnp.dot(newsB_ref[...], wn, preferred_element_type=jnp.float32) + bn)
    h0f = jnp.concatenate([ha, hb], axis=-1).reshape(BB, NF, 2 * DIM)
    ne = jnp.tanh(
        jnp.dot(cand_ref[...], wn, preferred_element_type=jnp.float32) + bn
    ).reshape(BB, NCAND, DIM)

    small = smallf_ref[...]
    hs = [h0f, small[0].reshape(BB, NF, 2 * DIM),
          small[1].reshape(BB, NF, 2 * DIM)]
    rs = [small[2].reshape(BB, NF, 2 * DIM), small[3].reshape(BB, NF, 2 * DIM),
          small[4].reshape(BB, NF, 2 * DIM)]
    ts = [small[5].reshape(BB, NF, 2 * DIM), small[6].reshape(BB, NF, 2 * DIM),
          small[7].reshape(BB, NF, 2 * DIM)]

    lane5 = lax.broadcasted_iota(jnp.int32, (BB, NCAND, 2 * DIM), 2)

    def softmax_lists(ps):
        mx = ps[0]
        for p in ps[1:]:
            mx = jnp.maximum(mx, p)
        es = [jnp.exp(p - mx) for p in ps]
        den = es[0]
        for e_ in es[1:]:
            den = den + e_
        return [e_ / den for e_ in es]

    o_sum = jnp.zeros((BB, NCAND, DIM), jnp.float32)
    for hop in range(NHOP + 1):
        Rh = rs[hop] * hs[hop]
        ne2 = jnp.concatenate([ne, ne], axis=-1)
        ne2lo = jnp.where(lane5 < DIM, ne2, 0.0)
        ne2hi = jnp.where(lane5 < DIM, 0.0, ne2)
        pl_ = [jnp.sum(Rh * ne2lo[:, c][:, None, :], axis=-1, keepdims=True)
               for c in range(NCAND)]
        ph_ = [jnp.sum(Rh * ne2hi[:, c][:, None, :], axis=-1, keepdims=True)
               for c in range(NCAND)]
        pnl = softmax_lists(pl_)
        pnh = softmax_lists(ph_)
        o = []
        for c in range(NCAND):
            osl = jnp.sum(ts[hop] * pnl[c], axis=1)
            osh = jnp.sum(ts[hop] * pnh[c], axis=1)
            o.append(osl[:, :DIM] + osh[:, DIM:])
        o = jnp.stack(o, axis=1)
        ne = jnp.dot((ne + o).reshape(BB * NCAND, DIM), wt,
                     preferred_element_type=jnp.float32).reshape(BB, NCAND, DIM)
        o_sum = o_sum + o

    scores = jnp.sum(ne * o_sum, axis=-1)
    scores_ref[...] = scores

    lab = labels_ref[...]
    cidx = lax.broadcasted_iota(jnp.int32, (BB, NCAND), 1)
    lmax = jnp.max(lab, axis=1, keepdims=True)
    tgt = jnp.min(jnp.where(lab >= lmax, cidx, NCAND), axis=1, keepdims=True)
    sc_t = jnp.sum(jnp.where(cidx == tgt, scores, 0.0), axis=1)
    smax = jnp.max(scores, axis=1)
    lse = smax + jnp.log(jnp.sum(jnp.exp(scores - smax[:, None]), axis=1))
    nll = jnp.sum(lse - sc_t)

    def sig_sum(x):
        return jnp.sum(1.0 / (1.0 + jnp.exp(-x)))

    kge0 = sig_sum(hs[0] * rs[0] * ts[0])
    kge1 = sig_sum(hs[1] * rs[1] * ts[1])
    l2 = (jnp.sum(hs[0] * hs[0]) + jnp.sum(hs[1] * hs[1])
          + jnp.sum(rs[0] * rs[0]) + jnp.sum(rs[1] * rs[1])
          + jnp.sum(ts[0] * ts[0]) + jnp.sum(ts[1] * ts[1]))

    row = lax.broadcasted_iota(jnp.int32, (8, 128), 0)
    lane = lax.broadcasted_iota(jnp.int32, (8, 128), 1)
    contrib = jnp.zeros((8, 128), jnp.float32)
    for k, v in enumerate([nll, kge0, kge1, l2]):
        contrib = contrib + jnp.where((row == k) & (lane == 0), v, 0.0)

    @pl.when(i == 0)
    def _init():
        acc_ref[...] = jnp.zeros((8, 128), jnp.float32)

    acc_ref[...] += contrib


_tc_compute = pl.pallas_call(
    _tc_body,
    grid=(GRID,),
    in_specs=[
        pl.BlockSpec((BB * NMEM // 2, TITLE), lambda i: (2 * i, 0)),
        pl.BlockSpec((BB * NMEM // 2, TITLE), lambda i: (2 * i + 1, 0)),
        pl.BlockSpec((BB * NCAND, TITLE), lambda i: (i, 0)),
        pl.BlockSpec((8, BB * NMEM // 2, 2 * DIM), lambda i: (0, i, 0)),
        pl.BlockSpec((BB, NCAND), lambda i: (i, 0)),
        pl.BlockSpec((TITLE, DIM), lambda i: (0, 0)),
        pl.BlockSpec((1, DIM), lambda i: (0, 0)),
        pl.BlockSpec((DIM, DIM), lambda i: (0, 0)),
    ],
    out_specs=[
        pl.BlockSpec((BB, NCAND), lambda i: (i, 0)),
        pl.BlockSpec((8, 128), lambda i: (0, 0)),
    ],
    out_shape=[
        jax.ShapeDtypeStruct((SB, NCAND), jnp.float32),
        jax.ShapeDtypeStruct((8, 128), jnp.float32),
    ],
)


def kernel(user_index, candidate_newsindex, labels, ripple_set, news_table,
           entity_table, relation_table, W_transform, W_n2e, b_n2e):
    ripple2d = ripple_set.reshape(NUSERS, 9 * NMEM).astype(jnp.int32)
    uidx = user_index.astype(jnp.int32)
    cand_flat = candidate_newsindex.reshape(B * NCAND).astype(jnp.int32)

    score_segs = []
    acc_sum = None
    for s in range(SEG):
        out_small, out_nidx = _get_k1()(
            lax.slice(uidx, (s * SB,), ((s + 1) * SB,)),
            ripple2d, entity_table, relation_table)
        out_news, out_cand = _get_k2()(
            out_nidx,
            lax.slice(cand_flat, (s * SB * NCAND,), ((s + 1) * SB * NCAND,)),
            news_table)
        smallf = out_small.reshape(8, SB * NMEM // 2, 2 * DIM)
        scores_s, acc_s = _tc_compute(
            out_news, out_news, out_cand, smallf,
            lax.slice(labels, (s * SB, 0), ((s + 1) * SB, NCAND)),
            W_n2e, b_n2e.reshape(1, DIM), W_transform)
        score_segs.append(scores_s)
        acc_sum = acc_s if acc_sum is None else acc_sum + acc_s

    scores = jnp.concatenate(score_segs, axis=0)
    acc = acc_sum

    col = acc[:, 0]
    denom = float(B * NMEM * DIM)
    base_loss = col[0] / B
    kge_loss = -KGE_W * (col[1] / denom + col[2] / denom)
    l2_loss = L2_W * col[3]
    loss = base_loss + kge_loss + l2_loss
    return (base_loss, kge_loss, l2_loss, loss, scores)


# speedup vs baseline: 1.0097x; 1.0097x over previous
import functools

import jax
import jax.numpy as jnp
from jax import lax
from jax.experimental import pallas as pl
from jax.experimental.pallas import tpu as pltpu
from jax.experimental.pallas import tpu_sc as plsc

B = 1024
NUSERS = 16384
NNEWS = 100001
DIM = 64
TITLE = 768
NHOP = 2
NMEM = 32
KGE_W = 0.01
L2_W = 1e-07

NC = 2
NS = 16
NW = NC * NS
SEG = 2
SB = B // SEG
UPT = SB // NW
NCAND = 5
CPT = NCAND * UPT
RPT = UPT * NMEM
CCH = CPT // NCAND

_ROLES = (
    ('ent', (1 * 3 + 0) * NMEM),
    ('ent', (2 * 3 + 0) * NMEM),
    ('rel', (0 * 3 + 1) * NMEM),
    ('rel', (1 * 3 + 1) * NMEM),
    ('rel', (2 * 3 + 1) * NMEM),
    ('ent', (0 * 3 + 2) * NMEM),
    ('ent', (1 * 3 + 2) * NMEM),
    ('ent', (2 * 3 + 2) * NMEM),
)

BB = 64
GRID = SB // BB
_GU = BB // UPT

ECH = 128
NECH = RPT // ECH
NCH = 64
NNCH = RPT // NCH


def _k1_body(uidx_hbm, ripple_hbm, er_hbm,
             out_small, out_nidx,
             uidx_v, rs_v, ridx_v, nidx_v, ebuf, sem_g, sem_w):
    wid = lax.axis_index("s") * NC + lax.axis_index("c")
    ubase = pl.multiple_of(wid * UPT, 8)

    pltpu.sync_copy(uidx_hbm.at[pl.ds(ubase, UPT)], uidx_v)
    pltpu.async_copy(ripple_hbm.at[uidx_v], rs_v, sem_g).wait()

    iota = lax.iota(jnp.int32, 16)
    def build(u, carry):
        uvec = jnp.full((16,), u, jnp.int32)
        for c in range(2):
            nidx_v[pl.ds(c * (RPT // 2) + u * 16, 16)] = \
                plsc.load_gather(rs_v, [uvec, 2 * iota + c])
            for p, (_, off) in enumerate(_ROLES):
                ridx_v[p, pl.ds(u * NMEM + c * 16, 16)] = \
                    rs_v[u, pl.ds(off + c * 16, 16)]
        return carry

    lax.fori_loop(0, UPT, build, 0)

    gu = _GU
    lob = pl.multiple_of(
        (wid // gu) * (BB * NMEM) + (wid % gu) * (UPT * NMEM // 2), 8)
    pltpu.sync_copy(nidx_v.at[pl.ds(0, RPT // 2)],
                    out_nidx.at[pl.ds(lob, RPT // 2)])
    pltpu.sync_copy(nidx_v.at[pl.ds(RPT // 2, RPT // 2)],
                    out_nidx.at[pl.ds(lob + BB * NMEM // 2, RPT // 2)])

    seq = [(p, ch) for p in range(8) for ch in range(NECH)]
    gd = {}
    wd = {}
    def half(p):
        return 0 if _ROLES[p][0] == 'ent' else 1

    def wr(j):
        p, ch = seq[j]
        return pltpu.async_copy(
            ebuf.at[j % 2, :, pl.ds(half(p) * DIM, DIM)],
            out_small.at[p, pl.ds(wid * RPT + ch * ECH, ECH)], sem_w)

    for j, (p, ch) in enumerate(seq):
        b = j % 2
        if j >= 2:
            wd[j - 2].wait()
        gd[j] = pltpu.async_copy(
            er_hbm.at[ridx_v.at[p, pl.ds(ch * ECH, ECH)]], ebuf.at[b], sem_g)
        if j >= 1:
            gd[j - 1].wait()
            wd[j - 1] = wr(j - 1)
    j = len(seq) - 1
    gd[j].wait()
    wd[j] = wr(j)
    wd[j - 1].wait()
    wd[j].wait()


def _k2_body(nidx_hbm, cand_hbm, news_hbm, out_news, out_cand,
             nidx_v, cidx_v, nbuf, sem_g, sem_w):
    wid = lax.axis_index("s") * NC + lax.axis_index("c")
    nbase = pl.multiple_of(wid * RPT, 8)
    cbase = pl.multiple_of(wid * CPT, 8)

    pltpu.sync_copy(nidx_hbm.at[pl.ds(nbase, RPT)], nidx_v)
    pltpu.sync_copy(cand_hbm.at[pl.ds(cbase, CPT)], cidx_v)

    seq = [('n', ch) for ch in range(NNCH)] + [('c', ch) for ch in range(NCAND)]

    def fire(j):
        kind, ch = seq[j]
        b = j % 2
        if kind == 'n':
            return pltpu.async_copy(
                news_hbm.at[nidx_v.at[pl.ds(ch * NCH, NCH)]],
                nbuf.at[b], sem_g)
        return pltpu.async_copy(
            news_hbm.at[cidx_v.at[pl.ds(ch * CCH, CCH)]],
            nbuf.at[b, pl.ds(0, CCH)], sem_g)

    def drain(j):
        kind, ch = seq[j]
        b = j % 2
        if kind == 'n':
            return pltpu.async_copy(
                nbuf.at[b], out_news.at[pl.ds(nbase + ch * NCH, NCH)], sem_w)
        return pltpu.async_copy(
            nbuf.at[b, pl.ds(0, CCH)],
            out_cand.at[pl.ds(cbase + ch * CCH, CCH)], sem_w)

    gd = {}
    wd = {}
    for j in range(len(seq)):
        if j >= 2:
            wd[j - 2].wait()
        gd[j] = fire(j)
        if j >= 1:
            gd[j - 1].wait()
            wd[j - 1] = drain(j - 1)
    j = len(seq) - 1
    gd[j].wait()
    wd[j] = drain(j)
    wd[j - 1].wait()
    wd[j].wait()


@functools.cache
def _get_k1():
    return functools.partial(
        pl.kernel,
        out_type=(
            jax.ShapeDtypeStruct((8, SB * NMEM, DIM), jnp.float32),
            jax.ShapeDtypeStruct((SB * NMEM,), jnp.int32),
        ),
        mesh=plsc.VectorSubcoreMesh(core_axis_name="c", subcore_axis_name="s",
                                    num_cores=NC, num_subcores=NS),
        scratch_types=[
            pltpu.VMEM((UPT,), jnp.int32),
            pltpu.VMEM((UPT, 9 * NMEM), jnp.int32),
            pltpu.VMEM((8, RPT), jnp.int32),
            pltpu.VMEM((RPT,), jnp.int32),
            pltpu.VMEM((2, ECH, 2 * DIM), jnp.float32),
            pltpu.SemaphoreType.DMA,
            pltpu.SemaphoreType.DMA,
        ],
        compiler_params=pltpu.CompilerParams(use_tc_tiling_on_sc=False,
                                             needs_layout_passes=False),
    )(_k1_body)


@functools.cache
def _get_k2():
    return functools.partial(
        pl.kernel,
        out_type=(
            jax.ShapeDtypeStruct((SB * NMEM, TITLE), jnp.float32),
            jax.ShapeDtypeStruct((SB * NCAND, TITLE), jnp.float32),
        ),
        mesh=plsc.VectorSubcoreMesh(core_axis_name="c", subcore_axis_name="s",
                                    num_cores=NC, num_subcores=NS),
        scratch_types=[
            pltpu.VMEM((RPT,), jnp.int32),
            pltpu.VMEM((CPT,), jnp.int32),
            pltpu.VMEM((2, NCH, TITLE), jnp.float32),
            pltpu.SemaphoreType.DMA,
            pltpu.SemaphoreType.DMA,
        ],
        compiler_params=pltpu.CompilerParams(use_tc_tiling_on_sc=True),
    )(_k2_body)


NF = NMEM // 2


def _tc_body(newsA_ref, newsB_ref, cand_ref, smallf_ref, labels_ref, wn_ref,
             bn_ref, wt_ref, scores_ref, acc_ref):
    i = pl.program_id(0)
    wn = wn_ref[...]
    bn = bn_ref[...]
    wt = wt_ref[...]

    ha = jnp.tanh(
        jnp.dot(newsA_ref[...], wn, preferred_element_type=jnp.float32) + bn)
    hb = jnp.tanh(
        jnp.dot(newsB_ref[...], wn, preferred_element_type=jnp.float32) + bn)
    h0f = jnp.concatenate([ha, hb], axis=-1).reshape(BB, NF, 2 * DIM)
    ne = jnp.tanh(
        jnp.dot(cand_ref[...], wn, preferred_element_type=jnp.float32) + bn
    ).reshape(BB, NCAND, DIM)

    small = smallf_ref[...]
    hs = [h0f, small[0].reshape(BB, NF, 2 * DIM),
          small[1].reshape(BB, NF, 2 * DIM)]
    rs = [small[2].reshape(BB, NF, 2 * DIM), small[3].reshape(BB, NF, 2 * DIM),
          small[4].reshape(BB, NF, 2 * DIM)]
    ts = [small[5].reshape(BB, NF, 2 * DIM), small[6].reshape(BB, NF, 2 * DIM),
          small[7].reshape(BB, NF, 2 * DIM)]

    lane5 = lax.broadcasted_iota(jnp.int32, (BB, NCAND, 2 * DIM), 2)

    def softmax_lists(ps):
        mx = ps[0]
        for p in ps[1:]:
            mx = jnp.maximum(mx, p)
        es = [jnp.exp(p - mx) for p in ps]
        den = es[0]
        for e_ in es[1:]:
            den = den + e_
        return [e_ / den for e_ in es]

    o_sum = jnp.zeros((BB, NCAND, DIM), jnp.float32)
    for hop in range(NHOP + 1):
        Rh = rs[hop] * hs[hop]
        ne2 = jnp.concatenate([ne, ne], axis=-1)
        ne2lo = jnp.where(lane5 < DIM, ne2, 0.0)
        ne2hi = jnp.where(lane5 < DIM, 0.0, ne2)
        pl_ = [jnp.sum(Rh * ne2lo[:, c][:, None, :], axis=-1, keepdims=True)
               for c in range(NCAND)]
        ph_ = [jnp.sum(Rh * ne2hi[:, c][:, None, :], axis=-1, keepdims=True)
               for c in range(NCAND)]
        pnl = softmax_lists(pl_)
        pnh = softmax_lists(ph_)
        o = []
        for c in range(NCAND):
            osl = jnp.sum(ts[hop] * pnl[c], axis=1)
            osh = jnp.sum(ts[hop] * pnh[c], axis=1)
            o.append(osl[:, :DIM] + osh[:, DIM:])
        o = jnp.stack(o, axis=1)
        ne = jnp.dot((ne + o).reshape(BB * NCAND, DIM), wt,
                     preferred_element_type=jnp.float32).reshape(BB, NCAND, DIM)
        o_sum = o_sum + o

    scores = jnp.sum(ne * o_sum, axis=-1)
    scores_ref[...] = scores

    lab = labels_ref[...]
    cidx = lax.broadcasted_iota(jnp.int32, (BB, NCAND), 1)
    lmax = jnp.max(lab, axis=1, keepdims=True)
    tgt = jnp.min(jnp.where(lab >= lmax, cidx, NCAND), axis=1, keepdims=True)
    sc_t = jnp.sum(jnp.where(cidx == tgt, scores, 0.0), axis=1)
    smax = jnp.max(scores, axis=1)
    lse = smax + jnp.log(jnp.sum(jnp.exp(scores - smax[:, None]), axis=1))
    nll = jnp.sum(lse - sc_t)

    def sig_sum(x):
        return jnp.sum(1.0 / (1.0 + jnp.exp(-x)))

    kge0 = sig_sum(hs[0] * rs[0] * ts[0])
    kge1 = sig_sum(hs[1] * rs[1] * ts[1])
    l2 = (jnp.sum(hs[0] * hs[0]) + jnp.sum(hs[1] * hs[1])
          + jnp.sum(rs[0] * rs[0]) + jnp.sum(rs[1] * rs[1])
          + jnp.sum(ts[0] * ts[0]) + jnp.sum(ts[1] * ts[1]))

    row = lax.broadcasted_iota(jnp.int32, (8, 128), 0)
    lane = lax.broadcasted_iota(jnp.int32, (8, 128), 1)
    contrib = jnp.zeros((8, 128), jnp.float32)
    for k, v in enumerate([nll, kge0, kge1, l2]):
        contrib = contrib + jnp.where((row == k) & (lane == 0), v, 0.0)

    @pl.when(i == 0)
    def _init():
        acc_ref[...] = jnp.zeros((8, 128), jnp.float32)

    acc_ref[...] += contrib


_tc_compute = pl.pallas_call(
    _tc_body,
    grid=(GRID,),
    in_specs=[
        pl.BlockSpec((BB * NMEM // 2, TITLE), lambda i: (2 * i, 0)),
        pl.BlockSpec((BB * NMEM // 2, TITLE), lambda i: (2 * i + 1, 0)),
        pl.BlockSpec((BB * NCAND, TITLE), lambda i: (i, 0)),
        pl.BlockSpec((8, BB * NMEM // 2, 2 * DIM), lambda i: (0, i, 0)),
        pl.BlockSpec((BB, NCAND), lambda i: (i, 0)),
        pl.BlockSpec((TITLE, DIM), lambda i: (0, 0)),
        pl.BlockSpec((1, DIM), lambda i: (0, 0)),
        pl.BlockSpec((DIM, DIM), lambda i: (0, 0)),
    ],
    out_specs=[
        pl.BlockSpec((BB, NCAND), lambda i: (i, 0)),
        pl.BlockSpec((8, 128), lambda i: (0, 0)),
    ],
    out_shape=[
        jax.ShapeDtypeStruct((SB, NCAND), jnp.float32),
        jax.ShapeDtypeStruct((8, 128), jnp.float32),
    ],
)


def kernel(user_index, candidate_newsindex, labels, ripple_set, news_table,
           entity_table, relation_table, W_transform, W_n2e, b_n2e):
    ripple2d = ripple_set.reshape(NUSERS, 9 * NMEM).astype(jnp.int32)
    er_table = jnp.concatenate([entity_table, relation_table], axis=1)
    uidx = user_index.astype(jnp.int32)
    cand_flat = candidate_newsindex.reshape(B * NCAND).astype(jnp.int32)

    score_segs = []
    acc_sum = None
    for s in range(SEG):
        out_small, out_nidx = _get_k1()(
            lax.slice(uidx, (s * SB,), ((s + 1) * SB,)),
            ripple2d, er_table)
        out_news, out_cand = _get_k2()(
            out_nidx,
            lax.slice(cand_flat, (s * SB * NCAND,), ((s + 1) * SB * NCAND,)),
            news_table)
        smallf = out_small.reshape(8, SB * NMEM // 2, 2 * DIM)
        scores_s, acc_s = _tc_compute(
            out_news, out_news, out_cand, smallf,
            lax.slice(labels, (s * SB, 0), ((s + 1) * SB, NCAND)),
            W_n2e, b_n2e.reshape(1, DIM), W_transform)
        score_segs.append(scores_s)
        acc_sum = acc_s if acc_sum is None else acc_sum + acc_s

    scores = jnp.concatenate(score_segs, axis=0)
    acc = acc_sum

    col = acc[:, 0]
    denom = float(B * NMEM * DIM)
    base_loss = col[0] / B
    kge_loss = -KGE_W * (col[1] / denom + col[2] / denom)
    l2_loss = L2_W * col[3]
    loss = base_loss + kge_loss + l2_loss
    return (base_loss, kge_loss, l2_loss, loss, scores)


# speedup vs baseline: 1.0138x; 1.0040x over previous
import functools

import jax
import jax.numpy as jnp
from jax import lax
from jax.experimental import pallas as pl
from jax.experimental.pallas import tpu as pltpu
from jax.experimental.pallas import tpu_sc as plsc

B = 1024
NUSERS = 16384
NNEWS = 100001
DIM = 64
TITLE = 768
NHOP = 2
NMEM = 32
KGE_W = 0.01
L2_W = 1e-07

NC = 2
NS = 16
NW = NC * NS
SEG = 2
SB = B // SEG
UPT = SB // NW
NCAND = 5
CPT = NCAND * UPT
RPT = UPT * NMEM
CCH = CPT // NCAND

_ROLES = (
    ('ent', (1 * 3 + 0) * NMEM),
    ('ent', (2 * 3 + 0) * NMEM),
    ('rel', (0 * 3 + 1) * NMEM),
    ('rel', (1 * 3 + 1) * NMEM),
    ('rel', (2 * 3 + 1) * NMEM),
    ('ent', (0 * 3 + 2) * NMEM),
    ('ent', (1 * 3 + 2) * NMEM),
    ('ent', (2 * 3 + 2) * NMEM),
)

BB = 64
GRID = SB // BB
_GU = BB // UPT

ECH = 128
NECH = RPT // ECH
NCH = 64
NNCH = RPT // NCH


def _k1_body(uidx_hbm, ripple_hbm, ent_hbm, rel_hbm,
             out_small, out_nidx,
             uidx_v, rs_v, ridx_v, nidx_v, ebuf, sem_g, sem_w):
    wid = lax.axis_index("s") * NC + lax.axis_index("c")
    ubase = pl.multiple_of(wid * UPT, 8)

    pltpu.sync_copy(uidx_hbm.at[pl.ds(ubase, UPT)], uidx_v)
    pltpu.async_copy(ripple_hbm.at[uidx_v], rs_v, sem_g).wait()

    iota = lax.iota(jnp.int32, 16)
    def build(u, carry):
        uvec = jnp.full((16,), u, jnp.int32)
        for c in range(2):
            nidx_v[pl.ds(c * (RPT // 2) + u * 16, 16)] = \
                plsc.load_gather(rs_v, [uvec, 2 * iota + c])
            for p, (_, off) in enumerate(_ROLES):
                ridx_v[p, pl.ds(u * NMEM + c * 16, 16)] = \
                    rs_v[u, pl.ds(off + c * 16, 16)]
        return carry

    lax.fori_loop(0, UPT, build, 0)

    gu = _GU
    lob = pl.multiple_of(
        (wid // gu) * (BB * NMEM) + (wid % gu) * (UPT * NMEM // 2), 8)
    pltpu.sync_copy(nidx_v.at[pl.ds(0, RPT // 2)],
                    out_nidx.at[pl.ds(lob, RPT // 2)])
    pltpu.sync_copy(nidx_v.at[pl.ds(RPT // 2, RPT // 2)],
                    out_nidx.at[pl.ds(lob + BB * NMEM // 2, RPT // 2)])

    seq = [(p, ch) for p in range(8) for ch in range(NECH)]
    gd = {}
    wd = {}
    def wr(j):
        p, ch = seq[j]
        return pltpu.async_copy(
            ebuf.at[j % 2],
            out_small.at[p, pl.ds(wid * RPT + ch * ECH, ECH)], sem_w)

    for j, (p, ch) in enumerate(seq):
        b = j % 2
        if j >= 2:
            wd[j - 2].wait()
        src = ent_hbm if _ROLES[p][0] == 'ent' else rel_hbm
        gd[j] = pltpu.async_copy(
            src.at[ridx_v.at[p, pl.ds(ch * ECH, ECH)]], ebuf.at[b], sem_g)
        if j >= 1:
            gd[j - 1].wait()
            wd[j - 1] = wr(j - 1)
    j = len(seq) - 1
    gd[j].wait()
    wd[j] = wr(j)
    wd[j - 1].wait()
    wd[j].wait()


def _k2_body(nidx_hbm, cand_hbm, news_hbm, out_news, out_cand,
             nidx_v, cidx_v, nbuf, sem_g, sem_w):
    wid = lax.axis_index("s") * NC + lax.axis_index("c")
    nbase = pl.multiple_of(wid * RPT, 8)
    cbase = pl.multiple_of(wid * CPT, 8)

    pltpu.sync_copy(nidx_hbm.at[pl.ds(nbase, RPT)], nidx_v)
    pltpu.sync_copy(cand_hbm.at[pl.ds(cbase, CPT)], cidx_v)

    seq = [('n', ch) for ch in range(NNCH)] + [('c', ch) for ch in range(NCAND)]

    def fire(j):
        kind, ch = seq[j]
        b = j % 2
        if kind == 'n':
            return pltpu.async_copy(
                news_hbm.at[nidx_v.at[pl.ds(ch * NCH, NCH)]],
                nbuf.at[b], sem_g)
        return pltpu.async_copy(
            news_hbm.at[cidx_v.at[pl.ds(ch * CCH, CCH)]],
            nbuf.at[b, pl.ds(0, CCH)], sem_g)

    def drain(j):
        kind, ch = seq[j]
        b = j % 2
        if kind == 'n':
            return pltpu.async_copy(
                nbuf.at[b], out_news.at[pl.ds(nbase + ch * NCH, NCH)], sem_w)
        return pltpu.async_copy(
            nbuf.at[b, pl.ds(0, CCH)],
            out_cand.at[pl.ds(cbase + ch * CCH, CCH)], sem_w)

    gd = {}
    wd = {}
    for j in range(len(seq)):
        if j >= 2:
            wd[j - 2].wait()
        gd[j] = fire(j)
        if j >= 1:
            gd[j - 1].wait()
            wd[j - 1] = drain(j - 1)
    j = len(seq) - 1
    gd[j].wait()
    wd[j] = drain(j)
    wd[j - 1].wait()
    wd[j].wait()


@functools.cache
def _get_k1():
    return functools.partial(
        pl.kernel,
        out_type=(
            jax.ShapeDtypeStruct((8, SB * NMEM, DIM), jnp.float32),
            jax.ShapeDtypeStruct((SB * NMEM,), jnp.int32),
        ),
        mesh=plsc.VectorSubcoreMesh(core_axis_name="c", subcore_axis_name="s",
                                    num_cores=NC, num_subcores=NS),
        scratch_types=[
            pltpu.VMEM((UPT,), jnp.int32),
            pltpu.VMEM((UPT, 9 * NMEM), jnp.int32),
            pltpu.VMEM((8, RPT), jnp.int32),
            pltpu.VMEM((RPT,), jnp.int32),
            pltpu.VMEM((2, ECH, DIM), jnp.float32),
            pltpu.SemaphoreType.DMA,
            pltpu.SemaphoreType.DMA,
        ],
        compiler_params=pltpu.CompilerParams(use_tc_tiling_on_sc=False,
                                             needs_layout_passes=False),
    )(_k1_body)


@functools.cache
def _get_k2():
    return functools.partial(
        pl.kernel,
        out_type=(
            jax.ShapeDtypeStruct((SB * NMEM, TITLE), jnp.float32),
            jax.ShapeDtypeStruct((SB * NCAND, TITLE), jnp.float32),
        ),
        mesh=plsc.VectorSubcoreMesh(core_axis_name="c", subcore_axis_name="s",
                                    num_cores=NC, num_subcores=NS),
        scratch_types=[
            pltpu.VMEM((RPT,), jnp.int32),
            pltpu.VMEM((CPT,), jnp.int32),
            pltpu.VMEM((2, NCH, TITLE), jnp.float32),
            pltpu.SemaphoreType.DMA,
            pltpu.SemaphoreType.DMA,
        ],
        compiler_params=pltpu.CompilerParams(use_tc_tiling_on_sc=True),
    )(_k2_body)


NF = NMEM // 2


def _tc_body(newsA_ref, newsB_ref, cand_ref, smallf_ref, labels_ref, wn_ref,
             bn_ref, wt_ref, scores_ref, acc_ref):
    i = pl.program_id(0)
    wn = wn_ref[...]
    bn = bn_ref[...]
    wt = wt_ref[...]

    ha = jnp.tanh(
        jnp.dot(newsA_ref[...], wn, preferred_element_type=jnp.float32) + bn)
    hb = jnp.tanh(
        jnp.dot(newsB_ref[...], wn, preferred_element_type=jnp.float32) + bn)
    h0f = jnp.concatenate([ha, hb], axis=-1).reshape(BB, NF, 2 * DIM)
    ne = jnp.tanh(
        jnp.dot(cand_ref[...], wn, preferred_element_type=jnp.float32) + bn
    ).reshape(BB, NCAND, DIM)

    small = smallf_ref[...]
    hs = [h0f, small[0].reshape(BB, NF, 2 * DIM),
          small[1].reshape(BB, NF, 2 * DIM)]
    rs = [small[2].reshape(BB, NF, 2 * DIM), small[3].reshape(BB, NF, 2 * DIM),
          small[4].reshape(BB, NF, 2 * DIM)]
    ts = [small[5].reshape(BB, NF, 2 * DIM), small[6].reshape(BB, NF, 2 * DIM),
          small[7].reshape(BB, NF, 2 * DIM)]

    lane5 = lax.broadcasted_iota(jnp.int32, (BB, NCAND, 2 * DIM), 2)

    def softmax_lists(ps):
        mx = ps[0]
        for p in ps[1:]:
            mx = jnp.maximum(mx, p)
        es = [jnp.exp(p - mx) for p in ps]
        den = es[0]
        for e_ in es[1:]:
            den = den + e_
        return [e_ / den for e_ in es]

    o_sum = jnp.zeros((BB, NCAND, DIM), jnp.float32)
    for hop in range(NHOP + 1):
        Rh = rs[hop] * hs[hop]
        ne2 = jnp.concatenate([ne, ne], axis=-1)
        ne2lo = jnp.where(lane5 < DIM, ne2, 0.0)
        ne2hi = jnp.where(lane5 < DIM, 0.0, ne2)
        pl_ = [jnp.sum(Rh * ne2lo[:, c][:, None, :], axis=-1, keepdims=True)
               for c in range(NCAND)]
        ph_ = [jnp.sum(Rh * ne2hi[:, c][:, None, :], axis=-1, keepdims=True)
               for c in range(NCAND)]
        pnl = softmax_lists(pl_)
        pnh = softmax_lists(ph_)
        o = []
        for c in range(NCAND):
            osl = jnp.sum(ts[hop] * pnl[c], axis=1)
            osh = jnp.sum(ts[hop] * pnh[c], axis=1)
            o.append(osl[:, :DIM] + osh[:, DIM:])
        o = jnp.stack(o, axis=1)
        ne = jnp.dot((ne + o).reshape(BB * NCAND, DIM), wt,
                     preferred_element_type=jnp.float32).reshape(BB, NCAND, DIM)
        o_sum = o_sum + o

    scores = jnp.sum(ne * o_sum, axis=-1)
    scores_ref[...] = scores

    lab = labels_ref[...]
    cidx = lax.broadcasted_iota(jnp.int32, (BB, NCAND), 1)
    lmax = jnp.max(lab, axis=1, keepdims=True)
    tgt = jnp.min(jnp.where(lab >= lmax, cidx, NCAND), axis=1, keepdims=True)
    sc_t = jnp.sum(jnp.where(cidx == tgt, scores, 0.0), axis=1)
    smax = jnp.max(scores, axis=1)
    lse = smax + jnp.log(jnp.sum(jnp.exp(scores - smax[:, None]), axis=1))
    nll = jnp.sum(lse - sc_t)

    def sig_sum(x):
        return jnp.sum(1.0 / (1.0 + jnp.exp(-x)))

    kge0 = sig_sum(hs[0] * rs[0] * ts[0])
    kge1 = sig_sum(hs[1] * rs[1] * ts[1])
    l2 = (jnp.sum(hs[0] * hs[0]) + jnp.sum(hs[1] * hs[1])
          + jnp.sum(rs[0] * rs[0]) + jnp.sum(rs[1] * rs[1])
          + jnp.sum(ts[0] * ts[0]) + jnp.sum(ts[1] * ts[1]))

    row = lax.broadcasted_iota(jnp.int32, (8, 128), 0)
    lane = lax.broadcasted_iota(jnp.int32, (8, 128), 1)
    contrib = jnp.zeros((8, 128), jnp.float32)
    for k, v in enumerate([nll, kge0, kge1, l2]):
        contrib = contrib + jnp.where((row == k) & (lane == 0), v, 0.0)

    @pl.when(i == 0)
    def _init():
        acc_ref[...] = jnp.zeros((8, 128), jnp.float32)

    acc_ref[...] += contrib


_tc_compute = pl.pallas_call(
    _tc_body,
    grid=(GRID,),
    in_specs=[
        pl.BlockSpec((BB * NMEM // 2, TITLE), lambda i: (2 * i, 0)),
        pl.BlockSpec((BB * NMEM // 2, TITLE), lambda i: (2 * i + 1, 0)),
        pl.BlockSpec((BB * NCAND, TITLE), lambda i: (i, 0)),
        pl.BlockSpec((8, BB * NMEM // 2, 2 * DIM), lambda i: (0, i, 0)),
        pl.BlockSpec((BB, NCAND), lambda i: (i, 0)),
        pl.BlockSpec((TITLE, DIM), lambda i: (0, 0)),
        pl.BlockSpec((1, DIM), lambda i: (0, 0)),
        pl.BlockSpec((DIM, DIM), lambda i: (0, 0)),
    ],
    out_specs=[
        pl.BlockSpec((BB, NCAND), lambda i: (i, 0)),
        pl.BlockSpec((8, 128), lambda i: (0, 0)),
    ],
    out_shape=[
        jax.ShapeDtypeStruct((SB, NCAND), jnp.float32),
        jax.ShapeDtypeStruct((8, 128), jnp.float32),
    ],
)


def kernel(user_index, candidate_newsindex, labels, ripple_set, news_table,
           entity_table, relation_table, W_transform, W_n2e, b_n2e):
    ripple2d = ripple_set.reshape(NUSERS, 9 * NMEM).astype(jnp.int32)
    uidx = user_index.astype(jnp.int32)
    cand_flat = candidate_newsindex.reshape(B * NCAND).astype(jnp.int32)

    score_segs = []
    acc_sum = None
    for s in range(SEG):
        out_small, out_nidx = _get_k1()(
            lax.slice(uidx, (s * SB,), ((s + 1) * SB,)),
            ripple2d, entity_table, relation_table)
        out_news, out_cand = _get_k2()(
            out_nidx,
            lax.slice(cand_flat, (s * SB * NCAND,), ((s + 1) * SB * NCAND,)),
            news_table)
        smallf = out_small.reshape(8, SB * NMEM // 2, 2 * DIM)
        scores_s, acc_s = _tc_compute(
            out_news, out_news, out_cand, smallf,
            lax.slice(labels, (s * SB, 0), ((s + 1) * SB, NCAND)),
            W_n2e, b_n2e.reshape(1, DIM), W_transform)
        score_segs.append(scores_s)
        acc_sum = acc_s if acc_sum is None else acc_sum + acc_s

    scores = jnp.concatenate(score_segs, axis=0)
    acc = acc_sum

    col = acc[:, 0]
    denom = float(B * NMEM * DIM)
    base_loss = col[0] / B
    kge_loss = -KGE_W * (col[1] / denom + col[2] / denom)
    l2_loss = L2_W * col[3]
    loss = base_loss + kge_loss + l2_loss
    return (base_loss, kge_loss, l2_loss, loss, scores)
